# baseline (speedup 1.0000x reference)
import functools

import jax
import jax.numpy as jnp
from jax import lax
from jax.experimental import pallas as pl
from jax.experimental.pallas import tpu as pltpu
from jax.experimental.pallas import tpu_sc as plsc

_NUM_CORES = 2
_NUM_SUBCORES = 16
_NW = _NUM_CORES * _NUM_SUBCORES

_CH = 8


def _gather_rows(flat_idx, table, *, B, ws2, c):
    b_per_w = B // _NW
    nch = b_per_w // _CH

    mesh = plsc.VectorSubcoreMesh(core_axis_name="c", subcore_axis_name="s")

    @functools.partial(
        pl.kernel,
        mesh=mesh,
        out_type=jax.ShapeDtypeStruct((B, ws2, c), jnp.float32),
        compiler_params=pltpu.CompilerParams(use_tc_tiling_on_sc=True),
        scratch_types=[
            pltpu.VMEM((b_per_w,), jnp.int32),
            pltpu.VMEM((2, _CH, ws2, c), jnp.float32),
            pltpu.SemaphoreType.DMA,
            pltpu.SemaphoreType.DMA,
            pltpu.SemaphoreType.DMA,
            pltpu.SemaphoreType.DMA,
        ],
    )
    def body(idx_hbm, table_hbm, out_hbm, idx_v, buf, g0, g1, s0, s1):
        wid = lax.axis_index("s") * _NUM_CORES + lax.axis_index("c")
        base = wid * b_per_w
        pltpu.sync_copy(idx_hbm.at[pl.ds(base, b_per_w)], idx_v)

        gsem = (g0, g1)
        ssem = (s0, s1)

        def issue_gather(i):
            p = i % 2
            return pltpu.async_copy(
                table_hbm.at[idx_v.at[pl.ds(i * _CH, _CH)]],
                buf.at[p],
                gsem[p],
            )

        def issue_write(i):
            p = i % 2
            return pltpu.async_copy(
                buf.at[p],
                out_hbm.at[pl.ds(base + i * _CH, _CH)],
                ssem[p],
            )

        g_next = issue_gather(0)
        w_prev = None
        for i in range(nch):
            g_cur = g_next
            if i + 1 < nch:
                if w_prev is not None:
                    w_prev.wait()
                g_next = issue_gather(i + 1)
            g_cur.wait()
            w_prev = issue_write(i)
        w_prev.wait()

    return body(flat_idx, table)


def kernel(index, image):
    b, mn, ws2, c = image.shape
    _, Nq, topw = index.shape
    D = ws2 * c
    B = b * Nq * topw

    table = image.reshape(b * mn, ws2, c)
    offs = (jnp.arange(b, dtype=jnp.int32) * mn)[:, None, None]
    flat_idx = (index.astype(jnp.int32) + offs).reshape(B)

    out = _gather_rows(flat_idx, table, B=B, ws2=ws2, c=c)
    return out.reshape(b, Nq, topw, ws2, c)


# speedup vs baseline: 1.8100x; 1.8100x over previous
import functools

import jax
import jax.numpy as jnp
from jax import lax
from jax.experimental import pallas as pl
from jax.experimental.pallas import tpu as pltpu
from jax.experimental.pallas import tpu_sc as plsc

_NUM_CORES = 2
_NUM_SUBCORES = 16
_NW = _NUM_CORES * _NUM_SUBCORES

_CH = 112


def _gather_rows(flat_idx, table, *, B, c):
    b_per_w = B // _NW
    nch = b_per_w // _CH
    assert nch % 2 == 0 and nch * _CH == b_per_w

    mesh = plsc.VectorSubcoreMesh(core_axis_name="c", subcore_axis_name="s")

    @functools.partial(
        pl.kernel,
        mesh=mesh,
        out_type=jax.ShapeDtypeStruct((B, c), jnp.float32),
        scratch_types=[
            pltpu.VMEM((b_per_w,), jnp.int32),
            pltpu.VMEM((2, _CH, c), jnp.float32),
            pltpu.SemaphoreType.DMA,
            pltpu.SemaphoreType.DMA,
            pltpu.SemaphoreType.DMA,
            pltpu.SemaphoreType.DMA,
        ],
    )
    def body(idx_hbm, table_hbm, out_hbm, idx_v, buf, g0, g1, s0, s1):
        wid = lax.axis_index("s") * _NUM_CORES + lax.axis_index("c")
        base = wid * b_per_w
        pltpu.sync_copy(idx_hbm.at[pl.ds(base, b_per_w)], idx_v)

        gsem = (g0, g1)
        ssem = (s0, s1)

        def gather(j, p):
            return pltpu.make_async_copy(
                table_hbm.at[idx_v.at[pl.ds(j * _CH, _CH)]],
                buf.at[p],
                gsem[p],
            )

        def write(j, p):
            return pltpu.make_async_copy(
                buf.at[p],
                out_hbm.at[pl.ds(base + j * _CH, _CH)],
                ssem[p],
            )

        gather(0, 0).start()
        gather(1, 1).start()

        @pl.loop(0, nch - 2, step=2)
        def _(j):
            gather(j, 0).wait()
            write(j, 0).start()
            gather(j + 1, 1).wait()
            write(j + 1, 1).start()
            write(j, 0).wait()
            gather(j + 2, 0).start()
            write(j + 1, 1).wait()
            gather(j + 3, 1).start()

        j = nch - 2
        gather(j, 0).wait()
        write(j, 0).start()
        gather(j + 1, 1).wait()
        write(j + 1, 1).start()
        write(j, 0).wait()
        write(j + 1, 1).wait()

    return body(flat_idx, table)


def kernel(index, image):
    b, mn, ws2, c = image.shape
    _, Nq, topw = index.shape
    B = b * Nq * ws2 * topw

    table = image.transpose(0, 2, 1, 3).reshape(b * ws2 * mn, c)

    idx_i32 = index.astype(jnp.int32)
    w_off = jnp.arange(ws2, dtype=jnp.int32) * mn
    b_off = jnp.arange(b, dtype=jnp.int32) * (ws2 * mn)
    g = (
        idx_i32[:, :, None, :]
        + w_off[None, None, :, None]
        + b_off[:, None, None, None]
    ).reshape(B)

    out = _gather_rows(g, table, B=B, c=c)
    out = out.reshape(b, Nq, ws2, topw, c).transpose(0, 1, 3, 2, 4)
    return out


# speedup vs baseline: 1.9378x; 1.0707x over previous
import functools

import jax
import jax.numpy as jnp
from jax import lax
from jax.experimental import pallas as pl
from jax.experimental.pallas import tpu as pltpu
from jax.experimental.pallas import tpu_sc as plsc

_NUM_CORES = 2
_NUM_SUBCORES = 16
_NW = _NUM_CORES * _NUM_SUBCORES

_CH = 112


def _gather_rows(flat_idx, table, *, B, c):
    b_per_w = B // _NW
    nch = b_per_w // _CH
    nbuf = 4
    assert nch % nbuf == 0 and nch * _CH == b_per_w

    mesh = plsc.VectorSubcoreMesh(core_axis_name="c", subcore_axis_name="s")

    @functools.partial(
        pl.kernel,
        mesh=mesh,
        out_type=jax.ShapeDtypeStruct((B, c), jnp.float32),
        scratch_types=[
            pltpu.VMEM((b_per_w,), jnp.int32),
            pltpu.VMEM((nbuf, _CH, c), jnp.float32),
            [pltpu.SemaphoreType.DMA] * nbuf,
            [pltpu.SemaphoreType.DMA] * nbuf,
        ],
    )
    def body(idx_hbm, table_hbm, out_hbm, idx_v, buf, gsem, ssem):
        wid = lax.axis_index("s") * _NUM_CORES + lax.axis_index("c")
        base = wid * b_per_w
        pltpu.sync_copy(idx_hbm.at[pl.ds(base, b_per_w)], idx_v)

        def gather(j, p):
            return pltpu.make_async_copy(
                table_hbm.at[idx_v.at[pl.ds(j * _CH, _CH)]],
                buf.at[p],
                gsem[p],
            )

        def write(j, p):
            return pltpu.make_async_copy(
                buf.at[p],
                out_hbm.at[pl.ds(base + j * _CH, _CH)],
                ssem[p],
            )

        for p in range(nbuf):
            gather(p, p).start()

        @pl.loop(0, nch - nbuf, step=nbuf)
        def _(j):
            for p in range(nbuf):
                gather(j + p, p).wait()
                write(j + p, p).start()
            for p in range(nbuf):
                write(j + p, p).wait()
                gather(j + nbuf + p, p).start()

        j = nch - nbuf
        for p in range(nbuf):
            gather(j + p, p).wait()
            write(j + p, p).start()
        for p in range(nbuf):
            write(j + p, p).wait()

    return body(flat_idx, table)


def kernel(index, image):
    b, mn, ws2, c = image.shape
    _, Nq, topw = index.shape
    B = b * Nq * ws2 * topw

    table = image.transpose(0, 2, 1, 3).reshape(b * ws2 * mn, c)

    idx_i32 = index.astype(jnp.int32)
    w_off = jnp.arange(ws2, dtype=jnp.int32) * mn
    b_off = jnp.arange(b, dtype=jnp.int32) * (ws2 * mn)
    g = (
        idx_i32[:, :, None, :]
        + w_off[None, None, :, None]
        + b_off[:, None, None, None]
    ).reshape(B)

    out = _gather_rows(g, table, B=B, c=c)
    out = out.reshape(b, Nq, ws2, topw, c).transpose(0, 1, 3, 2, 4)
    return out


# speedup vs baseline: 1.9409x; 1.0016x over previous
import functools

import jax
import jax.numpy as jnp
from jax import lax
from jax.experimental import pallas as pl
from jax.experimental.pallas import tpu as pltpu
from jax.experimental.pallas import tpu_sc as plsc

_NUM_CORES = 2
_NUM_SUBCORES = 16
_NW = _NUM_CORES * _NUM_SUBCORES

_CH = 112


def _gather_rows(flat_idx, table, *, B, c):
    b_per_w = B // _NW
    nch = b_per_w // _CH
    nbuf = 8
    assert nch % nbuf == 0 and nch * _CH == b_per_w

    mesh = plsc.VectorSubcoreMesh(core_axis_name="c", subcore_axis_name="s")

    @functools.partial(
        pl.kernel,
        mesh=mesh,
        out_type=jax.ShapeDtypeStruct((B, c), jnp.float32),
        scratch_types=[
            pltpu.VMEM((b_per_w,), jnp.int32),
            pltpu.VMEM((nbuf, _CH, c), jnp.float32),
            [pltpu.SemaphoreType.DMA] * nbuf,
            [pltpu.SemaphoreType.DMA] * nbuf,
        ],
    )
    def body(idx_hbm, table_hbm, out_hbm, idx_v, buf, gsem, ssem):
        wid = lax.axis_index("s") * _NUM_CORES + lax.axis_index("c")
        base = wid * b_per_w
        pltpu.sync_copy(idx_hbm.at[pl.ds(base, b_per_w)], idx_v)

        def gather(j, p):
            return pltpu.make_async_copy(
                table_hbm.at[idx_v.at[pl.ds(j * _CH, _CH)]],
                buf.at[p],
                gsem[p],
            )

        def write(j, p):
            return pltpu.make_async_copy(
                buf.at[p],
                out_hbm.at[pl.ds(base + j * _CH, _CH)],
                ssem[p],
            )

        for p in range(nbuf):
            gather(p, p).start()

        @pl.loop(0, nch - nbuf, step=nbuf)
        def _(j):
            for p in range(nbuf):
                gather(j + p, p).wait()
                write(j + p, p).start()
            for p in range(nbuf):
                write(j + p, p).wait()
                gather(j + nbuf + p, p).start()

        j = nch - nbuf
        for p in range(nbuf):
            gather(j + p, p).wait()
            write(j + p, p).start()
        for p in range(nbuf):
            write(j + p, p).wait()

    return body(flat_idx, table)


def kernel(index, image):
    b, mn, ws2, c = image.shape
    _, Nq, topw = index.shape
    B = b * Nq * ws2 * topw

    table = image.transpose(0, 2, 1, 3).reshape(b * ws2 * mn, c)

    idx_i32 = index.astype(jnp.int32)
    w_off = jnp.arange(ws2, dtype=jnp.int32) * mn
    b_off = jnp.arange(b, dtype=jnp.int32) * (ws2 * mn)
    g = (
        idx_i32[:, :, None, :]
        + w_off[None, None, :, None]
        + b_off[:, None, None, None]
    ).reshape(B)

    out = _gather_rows(g, table, B=B, c=c)
    out = out.reshape(b, Nq, ws2, topw, c).transpose(0, 1, 3, 2, 4)
    return out


# speedup vs baseline: 2.0825x; 1.0730x over previous
import functools

import jax
import jax.numpy as jnp
import numpy as np
from jax import lax
from jax.experimental import pallas as pl
from jax.experimental.pallas import tpu as pltpu
from jax.experimental.pallas import tpu_sc as plsc

_NUM_CORES = 2
_NUM_SUBCORES = 16
_NW = _NUM_CORES * _NUM_SUBCORES

_CH = 112
_NBUF = 4
_L = 16


def _gather_expand(idx_flat, table, *, B, c, mn, ws2, topw, q_per_w):
    b_per_w = B // _NW
    nch = b_per_w // _CH
    assert nch % _NBUF == 0 and nch * _CH == b_per_w
    rows_per_q = ws2 * topw
    grp = 2 * rows_per_q
    ngrp = b_per_w // grp
    nvreg = grp // _L

    m = np.arange(grp)
    jpat_np = (m // rows_per_q) * topw + (m % topw)
    woff_np = ((m % rows_per_q) // topw) * mn
    jpat_const = jnp.asarray(jpat_np, dtype=jnp.int32)
    woff_const = jnp.asarray(woff_np, dtype=jnp.int32)

    mesh = plsc.VectorSubcoreMesh(core_axis_name="c", subcore_axis_name="s")

    @functools.partial(
        pl.kernel,
        mesh=mesh,
        out_type=jax.ShapeDtypeStruct((B, c), jnp.float32),
        compiler_params=pltpu.CompilerParams(needs_layout_passes=False),
        scratch_types=[
            pltpu.VMEM((q_per_w * topw,), jnp.int32),
            pltpu.VMEM((grp,), jnp.int32),
            pltpu.VMEM((grp,), jnp.int32),
            pltpu.VMEM((b_per_w,), jnp.int32),
            pltpu.VMEM((_NBUF, _CH, c), jnp.float32),
            [pltpu.SemaphoreType.DMA] * _NBUF,
            [pltpu.SemaphoreType.DMA] * _NBUF,
        ],
    )
    def body(jpat_hbm, woff_hbm, idx_hbm, table_hbm, out_hbm,
             js_v, jpat_v, woff_v, idx_v, buf, gsem, ssem):
        wid = lax.axis_index("s") * _NUM_CORES + lax.axis_index("c")
        base = wid * b_per_w
        pltpu.sync_copy(idx_hbm.at[pl.ds(wid * q_per_w * topw, q_per_w * topw)], js_v)
        pltpu.sync_copy(jpat_hbm, jpat_v)
        pltpu.sync_copy(woff_hbm, woff_v)
        bconst = (wid // q_per_w) * (ws2 * mn)

        @pl.loop(0, ngrp)
        def _(qp):
            qoff = qp * (2 * topw)
            for k in range(nvreg):
                ji = jpat_v[pl.ds(k * _L, _L)] + qoff
                jv = plsc.load_gather(js_v, [ji])
                idx_v[pl.ds(qp * grp + k * _L, _L)] = (
                    jv + woff_v[pl.ds(k * _L, _L)] + bconst
                )

        def gather(j, p):
            return pltpu.make_async_copy(
                table_hbm.at[idx_v.at[pl.ds(j * _CH, _CH)]],
                buf.at[p],
                gsem[p],
            )

        def write(j, p):
            return pltpu.make_async_copy(
                buf.at[p],
                out_hbm.at[pl.ds(base + j * _CH, _CH)],
                ssem[p],
            )

        for p in range(_NBUF):
            gather(p, p).start()

        @pl.loop(0, nch - _NBUF, step=_NBUF)
        def _(j):
            for p in range(_NBUF):
                gather(j + p, p).wait()
                write(j + p, p).start()
            for p in range(_NBUF):
                write(j + p, p).wait()
                gather(j + _NBUF + p, p).start()

        j = nch - _NBUF
        for p in range(_NBUF):
            gather(j + p, p).wait()
            write(j + p, p).start()
        for p in range(_NBUF):
            write(j + p, p).wait()

    return body(jpat_const, woff_const, idx_flat, table)


def kernel(index, image):
    b, mn, ws2, c = image.shape
    _, Nq, topw = index.shape
    B = b * Nq * ws2 * topw

    table = image.transpose(0, 2, 1, 3).reshape(b * ws2 * mn, c)
    idx_flat = index.astype(jnp.int32).reshape(b * Nq * topw)

    q_per_w = (b * Nq) // _NW
    out = _gather_expand(
        idx_flat, table, B=B, c=c, mn=mn, ws2=ws2, topw=topw, q_per_w=q_per_w
    )
    out = out.reshape(b, Nq, ws2, topw, c).transpose(0, 1, 3, 2, 4)
    return out


# speedup vs baseline: 2.1987x; 1.0558x over previous
import functools

import jax
import jax.numpy as jnp
from jax import lax
from jax.experimental import pallas as pl
from jax.experimental.pallas import tpu as pltpu
from jax.experimental.pallas import tpu_sc as plsc

_NUM_CORES = 2
_NUM_SUBCORES = 16
_NW = _NUM_CORES * _NUM_SUBCORES

_CH = 112
_NBUF = 4
_L = 16


def _gather_expand(idx_flat, table, *, B, c, mn, ws2, topw, q_per_w):
    b_per_w = B // _NW
    nch = b_per_w // _CH
    assert nch % _NBUF == 0 and nch * _CH == b_per_w
    rows_per_q = ws2 * topw
    grp = 2 * rows_per_q
    ngrp = b_per_w // grp
    nvreg = grp // _L
    niter = (nch - _NBUF) // _NBUF

    mesh = plsc.VectorSubcoreMesh(core_axis_name="c", subcore_axis_name="s")

    @functools.partial(
        pl.kernel,
        mesh=mesh,
        out_type=jax.ShapeDtypeStruct((B, c), jnp.float32),
        compiler_params=pltpu.CompilerParams(needs_layout_passes=False),
        scratch_types=[
            pltpu.VMEM((q_per_w * topw,), jnp.int32),
            pltpu.VMEM((grp,), jnp.int32),
            pltpu.VMEM((grp,), jnp.int32),
            pltpu.VMEM((b_per_w,), jnp.int32),
            pltpu.VMEM((_NBUF, _CH, c), jnp.float32),
            [pltpu.SemaphoreType.DMA] * _NBUF,
            [pltpu.SemaphoreType.DMA] * _NBUF,
        ],
    )
    def body(idx_hbm, table_hbm, out_hbm,
             js_v, jpat_v, woff_v, idx_v, buf, gsem, ssem):
        wid = lax.axis_index("s") * _NUM_CORES + lax.axis_index("c")
        base = wid * b_per_w
        pltpu.sync_copy(idx_hbm.at[pl.ds(wid * q_per_w * topw, q_per_w * topw)], js_v)
        bconst = (wid // q_per_w) * (ws2 * mn)

        lane = jax.lax.iota(jnp.int32, _L)
        for k in range(nvreg):
            m = lane + (k * _L)
            hi = jnp.where(m >= rows_per_q, 1, 0)
            jpat_v[pl.ds(k * _L, _L)] = hi * topw + lane % topw
            woff_v[pl.ds(k * _L, _L)] = (
                (m - hi * rows_per_q) // topw
            ) * mn

        def expand(g):
            qoff = g * (2 * topw)
            for k in range(nvreg):
                ji = jpat_v[pl.ds(k * _L, _L)] + qoff
                jv = plsc.load_gather(js_v, [ji])
                idx_v[pl.ds(g * grp + k * _L, _L)] = (
                    jv + woff_v[pl.ds(k * _L, _L)] + bconst
                )

        def gather(j, p):
            return pltpu.make_async_copy(
                table_hbm.at[idx_v.at[pl.ds(j * _CH, _CH)]],
                buf.at[p],
                gsem[p],
            )

        def write(j, p):
            return pltpu.make_async_copy(
                buf.at[p],
                out_hbm.at[pl.ds(base + j * _CH, _CH)],
                ssem[p],
            )

        expand(0)
        expand(1)
        for p in range(_NBUF):
            gather(p, p).start()

        @pl.loop(0, niter)
        def _(i):
            j = i * _NBUF
            @pl.when(i < ngrp - 2)
            def _():
                expand(i + 2)
            for p in range(_NBUF):
                gather(j + p, p).wait()
                write(j + p, p).start()
            for p in range(_NBUF):
                write(j + p, p).wait()
                gather(j + _NBUF + p, p).start()

        j = nch - _NBUF
        for p in range(_NBUF):
            gather(j + p, p).wait()
            write(j + p, p).start()
        for p in range(_NBUF):
            write(j + p, p).wait()

    return body(idx_flat, table)


def kernel(index, image):
    b, mn, ws2, c = image.shape
    _, Nq, topw = index.shape
    B = b * Nq * ws2 * topw

    table = image.transpose(0, 2, 1, 3).reshape(b * ws2 * mn, c)
    idx_flat = index.astype(jnp.int32).reshape(b * Nq * topw)

    q_per_w = (b * Nq) // _NW
    out = _gather_expand(
        idx_flat, table, B=B, c=c, mn=mn, ws2=ws2, topw=topw, q_per_w=q_per_w
    )
    out = out.reshape(b, Nq, ws2, topw, c).transpose(0, 1, 3, 2, 4)
    return out
